# baseline (speedup 1.0000x reference)
import functools

import jax
import jax.numpy as jnp
from jax import lax
from jax.experimental import pallas as pl
from jax.experimental.pallas import tpu as pltpu
from jax.experimental.pallas import tpu_sc as plsc

NUM_EMBEDDINGS = 100000
EMBEDDING_DIM = 128
BATCH = 4096
FIELDS = 26

_NC = 2
_NS = 16
_NW = _NC * _NS

_B_PER_W = BATCH // _NW
_BC = 4
_C = _BC * FIELDS
_NCHUNK = _B_PER_W // _BC

_NBUF = 6
_LOOKAHEAD = 4


def _make_sc_gather():
  mesh = plsc.VectorSubcoreMesh(core_axis_name="c", subcore_axis_name="s")

  bufs = [pltpu.VMEM((_C, EMBEDDING_DIM), jnp.float32) for _ in range(_NBUF)]
  gsems = [pltpu.SemaphoreType.DMA for _ in range(_NBUF)]
  ssems = [pltpu.SemaphoreType.DMA for _ in range(_NBUF)]

  @functools.partial(
      pl.kernel,
      mesh=mesh,
      out_type=jax.ShapeDtypeStruct((BATCH, FIELDS, EMBEDDING_DIM),
                                    jnp.float32),
      compiler_params=pltpu.CompilerParams(use_tc_tiling_on_sc=True),
      scratch_types=[pltpu.VMEM((_NCHUNK, _C), jnp.int32)] + bufs + gsems + ssems,
  )
  def sc_gather(idx_hbm, table_hbm, out_hbm, idx_v, *scratch):
    buf = scratch[:_NBUF]
    gsem = scratch[_NBUF:2 * _NBUF]
    ssem = scratch[2 * _NBUF:]
    wid = lax.axis_index("s") * _NC + lax.axis_index("c")
    base_b = wid * _B_PER_W
    pltpu.sync_copy(idx_hbm.at[wid], idx_v)

    gathers = {}
    scatters = {}

    def start_gather(g):
      return pltpu.async_copy(
          table_hbm.at[idx_v.at[g]], buf[g % _NBUF], gsem[g % _NBUF])

    def start_scatter(g):
      b = buf[g % _NBUF]
      sem = ssem[g % _NBUF]
      last = None
      for j in range(_BC):
        last = pltpu.async_copy(
            b.at[pl.ds(j * FIELDS, FIELDS)],
            out_hbm.at[base_b + g * _BC + j], sem)
      return last

    def wait_scatter(g):
      for _ in range(_BC):
        scatters[g].wait()

    for g in range(-_LOOKAHEAD, _NCHUNK):
      ng = g + _LOOKAHEAD
      if ng < _NCHUNK:
        prev = ng - _NBUF
        if prev >= 0:
          wait_scatter(prev)
        gathers[ng] = start_gather(ng)
      if g >= 0:
        gathers[g].wait()
        scatters[g] = start_scatter(g)
    for g in range(max(0, _NCHUNK - _NBUF), _NCHUNK):
      wait_scatter(g)

  return sc_gather


_sc_gather = _make_sc_gather()


@jax.jit
def kernel(input, weight):
  idx = input.astype(jnp.int32).reshape(_NW, _NCHUNK, _C)
  return _sc_gather(idx, weight)


# speedup vs baseline: 1.8813x; 1.8813x over previous
import functools

import jax
import jax.numpy as jnp
from jax import lax
from jax.experimental import pallas as pl
from jax.experimental.pallas import tpu as pltpu
from jax.experimental.pallas import tpu_sc as plsc

NUM_EMBEDDINGS = 100000
EMBEDDING_DIM = 128
BATCH = 4096
FIELDS = 26

_NC = 2
_NS = 16
_NW = _NC * _NS

_C = 128
_NCHUNK = BATCH * FIELDS // (_NW * _C)
_CHUNKS_PER_F = BATCH // _C

_NBUF = 6
_LOOKAHEAD = 4


def _make_sc_gather():
  mesh = plsc.VectorSubcoreMesh(core_axis_name="c", subcore_axis_name="s")

  bufs = [pltpu.VMEM((_C, EMBEDDING_DIM), jnp.float32) for _ in range(_NBUF)]
  gsems = [pltpu.SemaphoreType.DMA for _ in range(_NBUF)]
  ssems = [pltpu.SemaphoreType.DMA for _ in range(_NBUF)]

  @functools.partial(
      pl.kernel,
      mesh=mesh,
      out_type=jax.ShapeDtypeStruct((FIELDS, BATCH, EMBEDDING_DIM),
                                    jnp.float32),
      compiler_params=pltpu.CompilerParams(use_tc_tiling_on_sc=True),
      scratch_types=[pltpu.VMEM((_NCHUNK * _C,), jnp.int32)] + bufs + gsems + ssems,
  )
  def sc_gather(idx_hbm, table_hbm, out_hbm, idx_v, *scratch):
    buf = scratch[:_NBUF]
    gsem = scratch[_NBUF:2 * _NBUF]
    ssem = scratch[2 * _NBUF:]
    wid = lax.axis_index("s") * _NC + lax.axis_index("c")
    c0 = wid * _NCHUNK
    pltpu.sync_copy(idx_hbm.at[wid], idx_v)

    gathers = {}
    scatters = {}

    def start_gather(k):
      return pltpu.async_copy(
          table_hbm.at[idx_v.at[pl.ds(k * _C, _C)]], buf[k % _NBUF],
          gsem[k % _NBUF])

    def start_scatter(k):
      c = c0 + k
      f = c // _CHUNKS_PER_F
      b0 = (c % _CHUNKS_PER_F) * _C
      return pltpu.async_copy(
          buf[k % _NBUF], out_hbm.at[f, pl.ds(b0, _C)], ssem[k % _NBUF])

    for k in range(-_LOOKAHEAD, _NCHUNK):
      nk = k + _LOOKAHEAD
      if nk < _NCHUNK:
        prev = nk - _NBUF
        if prev >= 0:
          scatters[prev].wait()
        gathers[nk] = start_gather(nk)
      if k >= 0:
        gathers[k].wait()
        scatters[k] = start_scatter(k)
    for k in range(max(0, _NCHUNK - _NBUF), _NCHUNK):
      scatters[k].wait()

  return sc_gather


_sc_gather = _make_sc_gather()


@jax.jit
def kernel(input, weight):
  idx = input.astype(jnp.int32).T.reshape(_NW, _NCHUNK * _C)
  out_fmajor = _sc_gather(idx, weight)
  return out_fmajor.transpose(1, 0, 2)
